# baseline (speedup 1.0000x reference)
import jax
import jax.numpy as jnp
from jax import lax
from jax.experimental import pallas as pl
from jax.experimental.pallas import tpu as pltpu
from jax.experimental.pallas import tpu_sc as plsc

N_ROWS_TABLE = 10000
D = 128
E = 320000
CH = 128
NCHUNKS = E // CH
NC, NS, L = 2, 16, 16
NW = NC * NS
BASE_CHUNKS = NCHUNKS // NW
EXTRA = NCHUNKS % NW


def _edge_dot_kernel(xm, xd, srcr, dstr, out, idx_a, idx_b, a_flat, b_flat,
                     out_v, sem):
    wid = lax.axis_index("s") * NC + lax.axis_index("c")
    nchunks = jnp.where(wid < EXTRA, BASE_CHUNKS + 1, BASE_CHUNKS)
    lanes = lax.iota(jnp.int32, L)

    def chunk_body(i, carry):
        r = wid + i * NW
        pltpu.sync_copy(srcr.at[r], idx_a)
        pltpu.sync_copy(dstr.at[r], idx_b)
        ha = pltpu.async_copy(xm.at[idx_a], a_flat, sem)
        hb = pltpu.async_copy(xd.at[idx_b], b_flat, sem)
        ha.wait()
        hb.wait()

        def group_body(g, c2):
            rowv = g * L + lanes
            acc = jnp.zeros((L,), jnp.float32)
            for d in range(D):
                col = jnp.full((L,), d, jnp.int32)
                av = plsc.load_gather(a_flat, [rowv, col])
                bv = plsc.load_gather(b_flat, [rowv, col])
                acc = acc + av * bv
            out_v[pl.ds(g * L, L)] = acc
            return c2

        lax.fori_loop(0, CH // L, group_body, 0, unroll=False)
        pltpu.sync_copy(out_v, out.at[pl.ds(r * CH, CH)])
        return carry

    lax.fori_loop(0, nchunks, chunk_body, 0, unroll=False)


def kernel(x_miRNA, x_disease, edge_label_index):
    eli = edge_label_index.astype(jnp.int32)
    src_r = eli[0].reshape(NCHUNKS, CH)
    dst_r = eli[1].reshape(NCHUNKS, CH)

    mesh = plsc.VectorSubcoreMesh(core_axis_name="c", subcore_axis_name="s")
    f = pl.kernel(
        _edge_dot_kernel,
        out_type=jax.ShapeDtypeStruct((E,), jnp.float32),
        mesh=mesh,
        scratch_types=[
            pltpu.VMEM((CH,), jnp.int32),
            pltpu.VMEM((CH,), jnp.int32),
            pltpu.VMEM((CH, D), jnp.float32),
            pltpu.VMEM((CH, D), jnp.float32),
            pltpu.VMEM((CH,), jnp.float32),
            pltpu.SemaphoreType.DMA,
        ],
        compiler_params=pltpu.CompilerParams(needs_layout_passes=False),
    )
    return f(x_miRNA, x_disease, src_r, dst_r)


# speedup vs baseline: 2.7847x; 2.7847x over previous
import jax
import jax.numpy as jnp
from jax import lax
from jax.experimental import pallas as pl
from jax.experimental.pallas import tpu as pltpu
from jax.experimental.pallas import tpu_sc as plsc

N_ROWS_TABLE = 10000
BITREV = (0, 8, 4, 12, 2, 10, 6, 14, 1, 9, 5, 13, 3, 11, 7, 15)
_GATHER_DNUMS = lax.GatherDimensionNumbers(
    offset_dims=(), collapsed_slice_dims=(0,), start_index_map=(0,))


def _perm16(x, s):
    idx = jnp.arange(16, dtype=jnp.int32) ^ s
    return lax.gather(x, idx[:, None], _GATHER_DNUMS, (1,),
                      mode=lax.GatherScatterMode.PROMISE_IN_BOUNDS)
D = 128
E = 320000
CH = 128
NCHUNKS = E // CH
NC, NS, L = 2, 16, 16
NW = NC * NS
BASE_CHUNKS = NCHUNKS // NW
EXTRA = NCHUNKS % NW


def _edge_dot_kernel(xm, xd, srcr, dstr, out, idx_a, idx_b, a_flat, b_flat,
                     out_v, sem):
    wid = lax.axis_index("s") * NC + lax.axis_index("c")
    nchunks = jnp.where(wid < EXTRA, BASE_CHUNKS + 1, BASE_CHUNKS)
    lanes = lax.iota(jnp.int32, L)

    def chunk_body(i, carry):
        r = wid + i * NW
        pltpu.sync_copy(srcr.at[r], idx_a)
        pltpu.sync_copy(dstr.at[r], idx_b)
        ha = pltpu.async_copy(xm.at[idx_a], a_flat, sem)
        hb = pltpu.async_copy(xd.at[idx_b], b_flat, sem)
        ha.wait()
        hb.wait()

        def group_body(g, c2):
            def combine(a, b, s):
                m = (lanes & s) == 0
                return (jnp.where(m, a, _perm16(b, s)) +
                        jnp.where(m, _perm16(a, s), b))

            pending = [None] * 5
            for j in range(L):
                e = g * L + BITREV[j]
                acc = a_flat[e, pl.ds(0, L)] * b_flat[e, pl.ds(0, L)]
                for k in range(1, D // L):
                    acc = acc + (a_flat[e, pl.ds(k * L, L)] *
                                 b_flat[e, pl.ds(k * L, L)])
                t = 0
                while pending[t] is not None:
                    acc = combine(pending[t], acc, 8 >> t)
                    pending[t] = None
                    t += 1
                pending[t] = acc
            out_v[pl.ds(g * L, L)] = pending[4]
            return c2

        lax.fori_loop(0, CH // L, group_body, 0, unroll=False)
        pltpu.sync_copy(out_v, out.at[pl.ds(r * CH, CH)])
        return carry

    lax.fori_loop(0, nchunks, chunk_body, 0, unroll=False)


def kernel(x_miRNA, x_disease, edge_label_index):
    eli = edge_label_index.astype(jnp.int32)
    src_r = eli[0].reshape(NCHUNKS, CH)
    dst_r = eli[1].reshape(NCHUNKS, CH)

    mesh = plsc.VectorSubcoreMesh(core_axis_name="c", subcore_axis_name="s")
    f = pl.kernel(
        _edge_dot_kernel,
        out_type=jax.ShapeDtypeStruct((E,), jnp.float32),
        mesh=mesh,
        scratch_types=[
            pltpu.VMEM((CH,), jnp.int32),
            pltpu.VMEM((CH,), jnp.int32),
            pltpu.VMEM((CH, D), jnp.float32),
            pltpu.VMEM((CH, D), jnp.float32),
            pltpu.VMEM((CH,), jnp.float32),
            pltpu.SemaphoreType.DMA,
        ],
        compiler_params=pltpu.CompilerParams(needs_layout_passes=False),
    )
    return f(x_miRNA, x_disease, src_r, dst_r)


# speedup vs baseline: 6.6998x; 2.4059x over previous
import jax
import jax.numpy as jnp
from jax import lax
from jax.experimental import pallas as pl
from jax.experimental.pallas import tpu as pltpu
from jax.experimental.pallas import tpu_sc as plsc

D = 128
E = 320000
CH = 128
NCHUNKS = E // CH
NC, NS, L = 2, 16, 16
NW = NC * NS
CPT = NCHUNKS // NW
PAIRS = CPT // 2
EXTRA = NCHUNKS % NW

BITREV = (0, 8, 4, 12, 2, 10, 6, 14, 1, 9, 5, 13, 3, 11, 7, 15)
_GATHER_DNUMS = lax.GatherDimensionNumbers(
    offset_dims=(), collapsed_slice_dims=(0,), start_index_map=(0,))


def _perm16(x, s):
    idx = jnp.arange(L, dtype=jnp.int32) ^ s
    return lax.gather(x, idx[:, None], _GATHER_DNUMS, (1,),
                      mode=lax.GatherScatterMode.PROMISE_IN_BOUNDS)


def _edge_dot_kernel(xm, xd, srcr, dstr, out, idx_a, idx_b, a0, b0, a1, b1,
                     out_v, sem0, sem1):
    wid = lax.axis_index("s") * NC + lax.axis_index("c")
    cbase = wid * CPT
    lanes = lax.iota(jnp.int32, L)

    def compute_chunk(c_local, a_rows, b_rows):
        def combine(a, b, s):
            m = (lanes & s) == 0
            return (jnp.where(m, a, _perm16(b, s)) +
                    jnp.where(m, _perm16(a, s), b))

        def group_body(g, c2):
            pending = [None] * 5
            for j in range(L):
                e = g * L + BITREV[j]
                acc = a_rows[e, pl.ds(0, L)] * b_rows[e, pl.ds(0, L)]
                for k in range(1, D // L):
                    acc = acc + (a_rows[e, pl.ds(k * L, L)] *
                                 b_rows[e, pl.ds(k * L, L)])
                t = 0
                while pending[t] is not None:
                    acc = combine(pending[t], acc, 8 >> t)
                    pending[t] = None
                    t += 1
                pending[t] = acc
            out_v[pl.ds(c_local * CH + g * L, L)] = pending[4]
            return c2

        lax.fori_loop(0, CH // L, group_body, 0, unroll=False)

    hia = pltpu.async_copy(srcr.at[pl.ds(cbase * CH, CPT * CH)], idx_a, sem0)
    hib = pltpu.async_copy(dstr.at[pl.ds(cbase * CH, CPT * CH)], idx_b, sem0)
    hia.wait()
    hib.wait()

    pltpu.async_copy(xm.at[idx_a.at[pl.ds(0, CH)]], a0, sem0)
    pltpu.async_copy(xd.at[idx_b.at[pl.ds(0, CH)]], b0, sem0)

    def pair_body(p, carry):
        c0 = 2 * p
        c1 = 2 * p + 1
        h1a = pltpu.async_copy(xm.at[idx_a.at[pl.ds(c1 * CH, CH)]], a1, sem1)
        h1b = pltpu.async_copy(xd.at[idx_b.at[pl.ds(c1 * CH, CH)]], b1, sem1)
        pltpu.make_async_copy(xm.at[idx_a.at[pl.ds(c0 * CH, CH)]], a0, sem0).wait()
        pltpu.make_async_copy(xd.at[idx_b.at[pl.ds(c0 * CH, CH)]], b0, sem0).wait()
        compute_chunk(c0, a0, b0)

        @pl.when(p < PAIRS - 1)
        def _():
            pltpu.async_copy(xm.at[idx_a.at[pl.ds((c0 + 2) * CH, CH)]], a0, sem0)
            pltpu.async_copy(xd.at[idx_b.at[pl.ds((c0 + 2) * CH, CH)]], b0, sem0)

        h1a.wait()
        h1b.wait()
        compute_chunk(c1, a1, b1)
        return carry

    lax.fori_loop(0, PAIRS, pair_body, 0, unroll=False)

    pltpu.sync_copy(out_v, out.at[pl.ds(cbase * CH, CPT * CH)])

    @pl.when(wid < EXTRA)
    def _():
        rglob = NW * CPT + wid
        pltpu.sync_copy(srcr.at[pl.ds(rglob * CH, CH)], idx_a.at[pl.ds(0, CH)])
        pltpu.sync_copy(dstr.at[pl.ds(rglob * CH, CH)], idx_b.at[pl.ds(0, CH)])
        ha = pltpu.async_copy(xm.at[idx_a.at[pl.ds(0, CH)]], a0, sem0)
        hb = pltpu.async_copy(xd.at[idx_b.at[pl.ds(0, CH)]], b0, sem0)
        ha.wait()
        hb.wait()
        compute_chunk(0, a0, b0)
        pltpu.sync_copy(out_v.at[pl.ds(0, CH)], out.at[pl.ds(rglob * CH, CH)])


def kernel(x_miRNA, x_disease, edge_label_index):
    eli = edge_label_index.astype(jnp.int32)
    src_r = eli[0]
    dst_r = eli[1]

    mesh = plsc.VectorSubcoreMesh(core_axis_name="c", subcore_axis_name="s")
    f = pl.kernel(
        _edge_dot_kernel,
        out_type=jax.ShapeDtypeStruct((E,), jnp.float32),
        mesh=mesh,
        scratch_types=[
            pltpu.VMEM((CPT * CH,), jnp.int32),
            pltpu.VMEM((CPT * CH,), jnp.int32),
            pltpu.VMEM((CH, D), jnp.float32),
            pltpu.VMEM((CH, D), jnp.float32),
            pltpu.VMEM((CH, D), jnp.float32),
            pltpu.VMEM((CH, D), jnp.float32),
            pltpu.VMEM((CPT * CH,), jnp.float32),
            pltpu.SemaphoreType.DMA,
            pltpu.SemaphoreType.DMA,
        ],
        compiler_params=pltpu.CompilerParams(needs_layout_passes=False),
    )
    return f(x_miRNA, x_disease, src_r, dst_r)
